# baseline (speedup 1.0000x reference)
import jax
import jax.numpy as jnp
from jax import lax
from jax.experimental import pallas as pl
from jax.experimental.pallas import tpu as pltpu

_N = 32768
_D = 1024
_S = 16
_EPS = 1e-05
_BN = 1024
_NB = _N // _BN


def _seg_ln_kernel(x_ref, seg_ref, gain_ref, bias_ref, out_ref,
                   sum_ref, sq_ref, cnt_ref, fin_ref):
    i = pl.program_id(0)

    @pl.when(i == 0)
    def _init():
        sum_ref[...] = jnp.zeros_like(sum_ref)
        sq_ref[...] = jnp.zeros_like(sq_ref)
        cnt_ref[...] = jnp.zeros_like(cnt_ref)

    ids = seg_ref[0]
    iota = lax.broadcasted_iota(jnp.int32, (_S, _BN), 0)
    onehot_t = jnp.broadcast_to(ids, (_S, _BN)) == iota

    @pl.when(i < _NB)
    def _accumulate():
        x = x_ref[...]
        oh = onehot_t.astype(jnp.bfloat16)
        xb = x.astype(jnp.bfloat16)
        xsq = (x * x).astype(jnp.bfloat16)
        sum_ref[...] += jnp.dot(oh, xb, preferred_element_type=jnp.float32)
        sq_ref[...] += jnp.dot(oh, xsq, preferred_element_type=jnp.float32)
        cnt_ref[:, 0:1] += jnp.sum(
            onehot_t.astype(jnp.float32), axis=1, keepdims=True)

    @pl.when(i == _NB)
    def _finalize():
        seg_sum = jnp.sum(sum_ref[...], axis=1, keepdims=True)
        seg_sq = jnp.sum(sq_ref[...], axis=1, keepdims=True)
        cnt = cnt_ref[:, 0:1]
        denom = jnp.maximum(cnt * float(_D), 1.0)
        mean = seg_sum / denom
        var = jnp.maximum(seg_sq / denom - mean * mean, 0.0)
        rstd = lax.rsqrt(var + _EPS)
        fin_ref[:, 0:1] = rstd
        fin_ref[:, 1:2] = mean * rstd

    @pl.when(i >= _NB)
    def _normalize():
        x = x_ref[...]
        oh = onehot_t.astype(jnp.float32)
        rs = lax.dot_general(
            oh, fin_ref[:, 0:2],
            dimension_numbers=(((0,), (0,)), ((), ())),
            preferred_element_type=jnp.float32)
        t = x * rs[:, 0:1] - rs[:, 1:2]
        out_ref[...] = t * gain_ref[...] + bias_ref[...]


def kernel(inputs, segment_ids, gain, bias):
    seg = jnp.reshape(segment_ids.astype(jnp.int32), (_NB, 1, _BN))
    gain2 = jnp.reshape(gain, (1, _D))
    bias2 = jnp.reshape(bias, (1, _D))

    out = pl.pallas_call(
        _seg_ln_kernel,
        grid=(2 * _NB,),
        in_specs=[
            pl.BlockSpec((_BN, _D), lambda i: (i % _NB, 0)),
            pl.BlockSpec((1, 1, _BN), lambda i: (i % _NB, 0, 0)),
            pl.BlockSpec((1, _D), lambda i: (0, 0)),
            pl.BlockSpec((1, _D), lambda i: (0, 0)),
        ],
        out_specs=pl.BlockSpec((_BN, _D), lambda i: (jnp.maximum(i - _NB, 0), 0)),
        out_shape=jax.ShapeDtypeStruct((_N, _D), jnp.float32),
        scratch_shapes=[
            pltpu.VMEM((_S, _D), jnp.float32),
            pltpu.VMEM((_S, _D), jnp.float32),
            pltpu.VMEM((_S, 128), jnp.float32),
            pltpu.VMEM((_S, 128), jnp.float32),
        ],
        compiler_params=pltpu.CompilerParams(
            dimension_semantics=("arbitrary",),
        ),
    )(inputs, seg, gain2, bias2)
    return out


# speedup vs baseline: 1.0891x; 1.0891x over previous
import jax
import jax.numpy as jnp
from jax import lax
from jax.experimental import pallas as pl
from jax.experimental.pallas import tpu as pltpu

_N = 32768
_D = 1024
_S = 16
_EPS = 1e-05
_BN = 2048
_NB = _N // _BN


def _seg_ln_kernel(x_ref, seg_ref, gain_ref, bias_ref, out_ref,
                   sum_ref, sq_ref, cnt_ref, fin_ref):
    i = pl.program_id(0)

    @pl.when(i == 0)
    def _init():
        sum_ref[...] = jnp.zeros_like(sum_ref)
        sq_ref[...] = jnp.zeros_like(sq_ref)
        cnt_ref[...] = jnp.zeros_like(cnt_ref)

    ids = seg_ref[0]
    iota = lax.broadcasted_iota(jnp.int32, (_S, _BN), 0)
    onehot_t = jnp.broadcast_to(ids, (_S, _BN)) == iota

    @pl.when(i < _NB)
    def _accumulate():
        x = x_ref[...]
        oh = onehot_t.astype(jnp.bfloat16)
        xb = x.astype(jnp.bfloat16)
        xsq = (x * x).astype(jnp.bfloat16)
        sum_ref[...] += jnp.dot(oh, xb, preferred_element_type=jnp.float32)
        sq_ref[...] += jnp.dot(oh, xsq, preferred_element_type=jnp.float32)
        cnt_ref[:, 0:1] += jnp.sum(
            onehot_t.astype(jnp.float32), axis=1, keepdims=True)

    @pl.when(i == _NB)
    def _finalize():
        seg_sum = jnp.sum(sum_ref[...], axis=1, keepdims=True)
        seg_sq = jnp.sum(sq_ref[...], axis=1, keepdims=True)
        cnt = cnt_ref[:, 0:1]
        denom = jnp.maximum(cnt * float(_D), 1.0)
        mean = seg_sum / denom
        var = jnp.maximum(seg_sq / denom - mean * mean, 0.0)
        rstd = lax.rsqrt(var + _EPS)
        fin_ref[:, 0:1] = rstd
        fin_ref[:, 1:2] = mean * rstd

    @pl.when(i >= _NB)
    def _normalize():
        x = x_ref[...]
        oh = onehot_t.astype(jnp.float32)
        rs = lax.dot_general(
            oh, fin_ref[:, 0:2],
            dimension_numbers=(((0,), (0,)), ((), ())),
            preferred_element_type=jnp.float32)
        t = x * rs[:, 0:1] - rs[:, 1:2]
        out_ref[...] = t * gain_ref[...] + bias_ref[...]


def kernel(inputs, segment_ids, gain, bias):
    seg = jnp.reshape(segment_ids.astype(jnp.int32), (_NB, 1, _BN))
    gain2 = jnp.reshape(gain, (1, _D))
    bias2 = jnp.reshape(bias, (1, _D))

    out = pl.pallas_call(
        _seg_ln_kernel,
        grid=(2 * _NB,),
        in_specs=[
            pl.BlockSpec((_BN, _D), lambda i: (i % _NB, 0)),
            pl.BlockSpec((1, 1, _BN), lambda i: (i % _NB, 0, 0)),
            pl.BlockSpec((1, _D), lambda i: (0, 0)),
            pl.BlockSpec((1, _D), lambda i: (0, 0)),
        ],
        out_specs=pl.BlockSpec((_BN, _D), lambda i: (jnp.maximum(i - _NB, 0), 0)),
        out_shape=jax.ShapeDtypeStruct((_N, _D), jnp.float32),
        scratch_shapes=[
            pltpu.VMEM((_S, _D), jnp.float32),
            pltpu.VMEM((_S, _D), jnp.float32),
            pltpu.VMEM((_S, 128), jnp.float32),
            pltpu.VMEM((_S, 128), jnp.float32),
        ],
        compiler_params=pltpu.CompilerParams(
            dimension_semantics=("arbitrary",),
        ),
    )(inputs, seg, gain2, bias2)
    return out
